# baseline (speedup 1.0000x reference)
import functools

import jax
import jax.numpy as jnp
import numpy as np
from jax import lax
from jax.experimental import pallas as pl
from jax.experimental.pallas import tpu as pltpu
from jax.experimental.pallas import tpu_sc as plsc

NC = 2
NS = 16
NW = NC * NS
CHUNK = 128
GROUP = 8


def _sc_scatter_kernel(n_pad, d, cpw, aux_lo):
    mesh = plsc.VectorSubcoreMesh(core_axis_name="c", subcore_axis_name="s")
    rows_per_tile = n_pad // NS
    ngroups = cpw // GROUP

    @functools.partial(
        pl.kernel,
        out_type=jax.ShapeDtypeStruct((NC, n_pad, d), jnp.float32),
        mesh=mesh,
        scratch_types=[
            pltpu.VMEM_SHARED((n_pad, d), jnp.float32),
            pltpu.VMEM((2, GROUP, CHUNK), jnp.int32),
            pltpu.VMEM((2, GROUP, CHUNK), jnp.int32),
            pltpu.VMEM((2, CHUNK, d), jnp.float32),
            pltpu.SemaphoreType.DMA,
            pltpu.SemaphoreType.DMA,
            pltpu.SemaphoreType.DMA,
            pltpu.SemaphoreType.DMA,
            pltpu.SemaphoreType.DMA,
        ],
    )
    def sc_kernel(x_hbm, ei3_hbm, aux_hbm, zero_hbm, out_hbm,
                  h_sh, sidx, didx, rows, gsem0, gsem1, isem, ssem0, ssem1):
        gsems = (gsem0, gsem1)
        ssems = (ssem0, ssem1)
        c = lax.axis_index("c")
        s = lax.axis_index("s")
        wid = s * NC + c
        r0 = s * rows_per_tile
        pltpu.sync_copy(zero_hbm, h_sh.at[pl.ds(r0, rows_per_tile)])
        plsc.subcore_barrier()

        def load_group(g, slot, sync):
            cb = g * GROUP

            @pl.when(wid < aux_lo)
            def _real():
                off = wid * cpw + cb
                if sync:
                    pltpu.sync_copy(ei3_hbm.at[0, pl.ds(off, GROUP)],
                                    sidx.at[slot])
                    pltpu.sync_copy(ei3_hbm.at[1, pl.ds(off, GROUP)],
                                    didx.at[slot])
                else:
                    pltpu.async_copy(ei3_hbm.at[0, pl.ds(off, GROUP)],
                                     sidx.at[slot], isem)
                    pltpu.async_copy(ei3_hbm.at[1, pl.ds(off, GROUP)],
                                     didx.at[slot], isem)

            @pl.when(wid >= aux_lo)
            def _aux():
                off = (wid - aux_lo) * cpw + cb
                if sync:
                    pltpu.sync_copy(aux_hbm.at[0, pl.ds(off, GROUP)],
                                    sidx.at[slot])
                    pltpu.sync_copy(aux_hbm.at[1, pl.ds(off, GROUP)],
                                    didx.at[slot])
                else:
                    pltpu.async_copy(aux_hbm.at[0, pl.ds(off, GROUP)],
                                     sidx.at[slot], isem)
                    pltpu.async_copy(aux_hbm.at[1, pl.ds(off, GROUP)],
                                     didx.at[slot], isem)

        def wait_group(slot):
            pltpu.make_async_copy(ei3_hbm.at[0, pl.ds(0, GROUP)],
                                  sidx.at[slot], isem).wait()
            pltpu.make_async_copy(ei3_hbm.at[1, pl.ds(0, GROUP)],
                                  didx.at[slot], isem).wait()

        load_group(0, 0, True)
        pltpu.async_copy(x_hbm.at[sidx.at[0, 0]], rows.at[0], gsem0)

        def pair_body(pp, carry):
            for gslot in range(2):
                g = 2 * pp + gslot
                for b in range(GROUP):
                    j = g * GROUP + b

                    pslot = gslot if b >= 1 else 1 - gslot
                    pb = b - 1 if b >= 1 else GROUP - 1

                    @pl.when((j >= 1) & (j + 1 < cpw))
                    def _drain_prev_scatter():
                        pltpu.make_async_copy(rows.at[(b + 1) % 2],
                                              h_sh.at[didx.at[pslot, pb]],
                                              ssems[(b + 1) % 2]).wait()

                    if b == 0:
                        @pl.when(g + 1 < ngroups)
                        def _prefetch_group():
                            load_group(g + 1, 1 - gslot, False)

                    if b < GROUP - 1:
                        @pl.when(j + 1 < cpw)
                        def _fire_next_gather():
                            pltpu.async_copy(x_hbm.at[sidx.at[gslot, b + 1]],
                                             rows.at[(b + 1) % 2],
                                             gsems[(b + 1) % 2])
                    else:
                        @pl.when(j + 1 < cpw)
                        def _fire_next_gather_newgroup():
                            wait_group(1 - gslot)
                            pltpu.async_copy(x_hbm.at[sidx.at[1 - gslot, 0]],
                                             rows.at[(b + 1) % 2],
                                             gsems[(b + 1) % 2])

                    pltpu.make_async_copy(x_hbm.at[sidx.at[gslot, b]],
                                          rows.at[b % 2], gsems[b % 2]).wait()
                    pltpu.async_copy(rows.at[b % 2],
                                     h_sh.at[didx.at[gslot, b]],
                                     ssems[b % 2], add=True)
            return carry

        lax.fori_loop(0, ngroups // 2, pair_body, 0)
        ls = (ngroups - 1) % 2
        pltpu.make_async_copy(rows.at[0], h_sh.at[didx.at[ls, GROUP - 2]],
                              ssems[0]).wait()
        pltpu.make_async_copy(rows.at[1], h_sh.at[didx.at[ls, GROUP - 1]],
                              ssems[1]).wait()
        plsc.subcore_barrier()
        pltpu.sync_copy(h_sh.at[pl.ds(r0, rows_per_tile)],
                        out_hbm.at[c, pl.ds(r0, rows_per_tile)])

    return sc_kernel


def _tc_dense_kernel(p_ref, we_ref, be_ref, wd_ref, bd_ref, o_ref):
    h = p_ref[0] + p_ref[1]
    lat = jnp.dot(h, we_ref[...], preferred_element_type=jnp.float32)
    lat = jnp.maximum(lat + be_ref[...], 0.0)
    rec = jnp.dot(lat, wd_ref[...], preferred_element_type=jnp.float32)
    rec = rec + bd_ref[...]
    e = jnp.exp(rec)
    o_ref[...] = e / jnp.sum(e, axis=-1, keepdims=True)


def kernel(x, edge_index, W_enc, b_enc, W_dec, b_dec):
    n, d = x.shape
    e = edge_index.shape[1]
    lat_dim = W_enc.shape[1]

    n_pad = ((n + 8 * NS) + (128 * NS - 1)) // (128 * NS) * (128 * NS)
    cpw = -(-e // (NW * CHUNK))
    cpw = (cpw + 2 * GROUP - 1) // (2 * GROUP) * (2 * GROUP)
    n_chunks = cpw * NW
    full_chunks = e // CHUNK
    aux_lo = min(full_chunks // cpw, NW - 1)
    aux_chunks = n_chunks - aux_lo * cpw
    real_tail = e - aux_lo * cpw * CHUNK

    pad_len = aux_chunks * CHUNK - real_tail
    pad_src = np.arange(pad_len, dtype=np.int32) % n
    pad_dst = (n + np.arange(pad_len, dtype=np.int32) % (n_pad - n)).astype(
        np.int32)
    aux_src = jnp.concatenate(
        [edge_index[0, aux_lo * cpw * CHUNK:], jnp.asarray(pad_src)])
    aux_dst = jnp.concatenate(
        [edge_index[1, aux_lo * cpw * CHUNK:], jnp.asarray(pad_dst)])
    aux = jnp.stack([aux_src.reshape(aux_chunks, CHUNK),
                     aux_dst.reshape(aux_chunks, CHUNK)], axis=0)
    ei3 = edge_index[:, :full_chunks * CHUNK].reshape(2, full_chunks, CHUNK)
    zero = jnp.asarray(np.zeros((n_pad // NS, d), np.float32))

    partials = _sc_scatter_kernel(n_pad, d, cpw, aux_lo)(
        x, ei3, aux, zero)

    grid = 5
    br = n // grid
    prob = pl.pallas_call(
        _tc_dense_kernel,
        grid=(grid,),
        in_specs=[
            pl.BlockSpec((NC, br, d), lambda i: (0, i, 0)),
            pl.BlockSpec((d, lat_dim), lambda i: (0, 0)),
            pl.BlockSpec((1, lat_dim), lambda i: (0, 0)),
            pl.BlockSpec((lat_dim, d), lambda i: (0, 0)),
            pl.BlockSpec((1, d), lambda i: (0, 0)),
        ],
        out_specs=pl.BlockSpec((br, d), lambda i: (i, 0)),
        out_shape=jax.ShapeDtypeStruct((n, d), jnp.float32),
    )(partials, W_enc, b_enc.reshape(1, lat_dim), W_dec, b_dec.reshape(1, d))

    return prob


# speedup vs baseline: 1.0564x; 1.0564x over previous
import functools

import jax
import jax.numpy as jnp
import numpy as np
from jax import lax
from jax.experimental import pallas as pl
from jax.experimental.pallas import tpu as pltpu
from jax.experimental.pallas import tpu_sc as plsc

NC = 2
NS = 16
NW = NC * NS
CHUNK = 128


def _sc_scatter_kernel(n_pad, d, cpw, full_chunks):
    mesh = plsc.VectorSubcoreMesh(core_axis_name="c", subcore_axis_name="s")
    rows_per_tile = n_pad // NS

    @functools.partial(
        pl.kernel,
        out_type=jax.ShapeDtypeStruct((NC, n_pad, d), jnp.float32),
        mesh=mesh,
        scratch_types=[
            pltpu.VMEM_SHARED((n_pad, d), jnp.float32),
            pltpu.VMEM((4, CHUNK), jnp.int32),
            pltpu.VMEM((4, CHUNK), jnp.int32),
            pltpu.VMEM((2, CHUNK, d), jnp.float32),
            pltpu.SemaphoreType.DMA,
            pltpu.SemaphoreType.DMA,
            pltpu.SemaphoreType.DMA,
            pltpu.SemaphoreType.DMA,
            pltpu.SemaphoreType.DMA,
        ],
    )
    def sc_kernel(x_hbm, ei_hbm, aux_hbm, zero_hbm, out_hbm,
                  h_sh, sidx, didx, rows, gsem0, gsem1, isem, ssem0, ssem1):
        gsems = (gsem0, gsem1)
        ssems = (ssem0, ssem1)
        c = lax.axis_index("c")
        s = lax.axis_index("s")
        wid = s * NC + c
        r0 = s * rows_per_tile
        pltpu.sync_copy(zero_hbm, h_sh.at[pl.ds(r0, rows_per_tile)])
        plsc.subcore_barrier()

        def load_idx(j, slot, sync):
            cid = j * NW + wid

            @pl.when(cid < full_chunks)
            def _real():
                off = cid * CHUNK
                if sync:
                    pltpu.sync_copy(ei_hbm.at[0, pl.ds(off, CHUNK)],
                                    sidx.at[slot])
                    pltpu.sync_copy(ei_hbm.at[1, pl.ds(off, CHUNK)],
                                    didx.at[slot])
                else:
                    pltpu.async_copy(ei_hbm.at[0, pl.ds(off, CHUNK)],
                                     sidx.at[slot], isem)
                    pltpu.async_copy(ei_hbm.at[1, pl.ds(off, CHUNK)],
                                     didx.at[slot], isem)

            @pl.when(cid >= full_chunks)
            def _aux():
                a = cid - full_chunks
                if sync:
                    pltpu.sync_copy(aux_hbm.at[a, 0], sidx.at[slot])
                    pltpu.sync_copy(aux_hbm.at[a, 1], didx.at[slot])
                else:
                    pltpu.async_copy(aux_hbm.at[a, 0], sidx.at[slot], isem)
                    pltpu.async_copy(aux_hbm.at[a, 1], didx.at[slot], isem)

        def wait_idx(slot):
            pltpu.make_async_copy(ei_hbm.at[0, pl.ds(0, CHUNK)],
                                  sidx.at[slot], isem).wait()
            pltpu.make_async_copy(ei_hbm.at[1, pl.ds(0, CHUNK)],
                                  didx.at[slot], isem).wait()

        load_idx(0, 0, True)
        pltpu.async_copy(x_hbm.at[sidx.at[0]], rows.at[0], gsem0)
        load_idx(1, 1, False)

        def quad_body(p, carry):
            for b in range(4):
                j = 4 * p + b
                kn = (b + 1) % 4
                kf = (b + 2) % 4

                @pl.when((j >= 1) & (j + 1 < cpw))
                def _drain_prev_scatter():
                    pltpu.make_async_copy(rows.at[(b + 1) % 2],
                                          h_sh.at[didx.at[(b - 1) % 4]],
                                          ssems[(b + 1) % 2]).wait()

                @pl.when(j + 1 < cpw)
                def _ready_next_gather():
                    wait_idx(kn)
                    pltpu.async_copy(x_hbm.at[sidx.at[kn]],
                                     rows.at[(b + 1) % 2], gsems[(b + 1) % 2])

                @pl.when(j + 2 < cpw)
                def _prefetch_idx():
                    load_idx(j + 2, kf, False)

                pltpu.make_async_copy(x_hbm.at[sidx.at[b]],
                                      rows.at[b % 2], gsems[b % 2]).wait()
                pltpu.async_copy(rows.at[b % 2], h_sh.at[didx.at[b]],
                                 ssems[b % 2], add=True)
            return carry

        lax.fori_loop(0, cpw // 4, quad_body, 0)
        pltpu.make_async_copy(rows.at[(cpw - 2) % 2],
                              h_sh.at[didx.at[(cpw - 2) % 4]],
                              ssems[(cpw - 2) % 2]).wait()
        pltpu.make_async_copy(rows.at[(cpw - 1) % 2],
                              h_sh.at[didx.at[(cpw - 1) % 4]],
                              ssems[(cpw - 1) % 2]).wait()
        plsc.subcore_barrier()
        pltpu.sync_copy(h_sh.at[pl.ds(r0, rows_per_tile)],
                        out_hbm.at[c, pl.ds(r0, rows_per_tile)])

    return sc_kernel


def _tc_dense_kernel(p_ref, we_ref, be_ref, wd_ref, bd_ref, o_ref):
    h = p_ref[0] + p_ref[1]
    lat = jnp.dot(h, we_ref[...], preferred_element_type=jnp.float32)
    lat = jnp.maximum(lat + be_ref[...], 0.0)
    rec = jnp.dot(lat, wd_ref[...], preferred_element_type=jnp.float32)
    rec = rec + bd_ref[...]
    e = jnp.exp(rec)
    o_ref[...] = e / jnp.sum(e, axis=-1, keepdims=True)


def kernel(x, edge_index, W_enc, b_enc, W_dec, b_dec):
    n, d = x.shape
    e = edge_index.shape[1]
    lat_dim = W_enc.shape[1]

    n_pad = ((n + 8 * NS) + (128 * NS - 1)) // (128 * NS) * (128 * NS)
    cpw = -(-e // (NW * CHUNK))
    cpw = (cpw + 3) // 4 * 4
    n_chunks = cpw * NW
    full_chunks = e // CHUNK
    rem = e - full_chunks * CHUNK
    aux_cnt = n_chunks - full_chunks

    pad_len = aux_cnt * CHUNK - rem
    pad_src = np.arange(pad_len, dtype=np.int32) % n
    pad_dst = (n + np.arange(pad_len, dtype=np.int32) % (n_pad - n)).astype(
        np.int32)
    if rem:
        aux_src = jnp.concatenate(
            [edge_index[0, full_chunks * CHUNK:], jnp.asarray(pad_src)])
        aux_dst = jnp.concatenate(
            [edge_index[1, full_chunks * CHUNK:], jnp.asarray(pad_dst)])
        aux = jnp.stack(
            [aux_src.reshape(aux_cnt, CHUNK), aux_dst.reshape(aux_cnt, CHUNK)],
            axis=1)
    else:
        aux = jnp.asarray(
            np.stack([pad_src.reshape(aux_cnt, CHUNK),
                      pad_dst.reshape(aux_cnt, CHUNK)], axis=1))
    zero = jnp.asarray(np.zeros((n_pad // NS, d), np.float32))

    partials = _sc_scatter_kernel(n_pad, d, cpw, full_chunks)(
        x, edge_index, aux, zero)

    grid = 5
    br = n // grid
    prob = pl.pallas_call(
        _tc_dense_kernel,
        grid=(grid,),
        in_specs=[
            pl.BlockSpec((NC, br, d), lambda i: (0, i, 0)),
            pl.BlockSpec((d, lat_dim), lambda i: (0, 0)),
            pl.BlockSpec((1, lat_dim), lambda i: (0, 0)),
            pl.BlockSpec((lat_dim, d), lambda i: (0, 0)),
            pl.BlockSpec((1, d), lambda i: (0, 0)),
        ],
        out_specs=pl.BlockSpec((br, d), lambda i: (i, 0)),
        out_shape=jax.ShapeDtypeStruct((n, d), jnp.float32),
    )(partials, W_enc, b_enc.reshape(1, lat_dim), W_dec, b_dec.reshape(1, d))

    return prob
